# speedup vs baseline: 2.9874x; 2.9874x over previous
import functools

import jax
import jax.numpy as jnp
from jax import lax
from jax.experimental import pallas as pl
from jax.experimental.pallas import tpu as pltpu
from jax.experimental.pallas import tpu_sc as plsc

NC = 2
NS = 16
NW = NC * NS

B = 16384 * 26
D = 128
K = 128
B_PER_W = B // NW
NCHUNK = B_PER_W // K


def _body(idx_hbm, table_hbm, out_hbm, idx_v, rows_v, sem):
    wid = lax.axis_index("s") * NC + lax.axis_index("c")
    pltpu.sync_copy(idx_hbm.at[wid], idx_v)
    out_base = wid * B_PER_W

    def chunk(c, carry):
        pltpu.async_copy(table_hbm.at[idx_v.at[c]], rows_v, sem).wait()
        pltpu.sync_copy(rows_v, out_hbm.at[pl.ds(out_base + c * K, K)])
        return carry

    lax.fori_loop(0, NCHUNK, chunk, 0)


_gather_call = functools.partial(
    pl.kernel,
    out_type=jax.ShapeDtypeStruct((B, D), jnp.float32),
    mesh=plsc.VectorSubcoreMesh(core_axis_name="c", subcore_axis_name="s"),
    scratch_types=[
        pltpu.VMEM((NCHUNK, K), jnp.int32),
        pltpu.VMEM((K, D), jnp.float32),
        pltpu.SemaphoreType.DMA,
    ],
)(_body)


@jax.jit
def kernel(indices, embedding_table):
    idx = indices.reshape(NW, NCHUNK, K).astype(jnp.int32)
    out = _gather_call(idx, embedding_table)
    return out.reshape(16384, 26, D)


# speedup vs baseline: 3.3802x; 1.1315x over previous
import functools

import jax
import jax.numpy as jnp
from jax import lax
from jax.experimental import pallas as pl
from jax.experimental.pallas import tpu as pltpu
from jax.experimental.pallas import tpu_sc as plsc

NC = 2
NS = 16
NW = NC * NS

B = 16384 * 26
D = 128
K = 128
B_PER_W = B // NW
NCHUNK = B_PER_W // K
NBUF = 4
NROUND = NCHUNK // NBUF


def _body(idx_hbm, table_hbm, out_hbm, idx_v, rows_v, *sems):
    sems_g, sems_w = sems[:NBUF], sems[NBUF:]
    wid = lax.axis_index("s") * NC + lax.axis_index("c")
    pltpu.sync_copy(idx_hbm.at[wid], idx_v)
    out_base = wid * B_PER_W

    def start_gather(c, b):
        pltpu.async_copy(table_hbm.at[idx_v.at[c]], rows_v.at[b], sems_g[b])

    def wait_gather(b):
        pltpu.make_async_copy(
            table_hbm.at[idx_v.at[0]], rows_v.at[b], sems_g[b]).wait()

    def start_write(c, b):
        pltpu.async_copy(
            rows_v.at[b], out_hbm.at[pl.ds(out_base + c * K, K)], sems_w[b])

    def wait_write(b):
        pltpu.make_async_copy(
            rows_v.at[b], out_hbm.at[pl.ds(out_base, K)], sems_w[b]).wait()

    def visit(g, b, do_wait_w, do_gather):
        bw = (b + 2) % NBUF
        if do_wait_w:
            wait_write(bw)
        if do_gather:
            start_gather(g + 2, bw)
        wait_gather(b)
        start_write(g, b)

    start_gather(0, 0)
    start_gather(1, 1)

    visit(0, 0, False, True)
    visit(1, 1, False, True)
    visit(2, 2, True, True)
    visit(3, 3, True, True)

    def round_body(r, carry):
        g0 = r * NBUF
        for b in range(NBUF):
            visit(g0 + b, b, True, True)
        return carry

    lax.fori_loop(1, NROUND - 1, round_body, 0)

    g0 = (NROUND - 1) * NBUF
    visit(g0 + 0, 0, True, True)
    visit(g0 + 1, 1, True, True)
    visit(g0 + 2, 2, True, False)
    visit(g0 + 3, 3, True, False)

    wait_write(2)
    wait_write(3)


_gather_call = functools.partial(
    pl.kernel,
    out_type=jax.ShapeDtypeStruct((B, D), jnp.float32),
    mesh=plsc.VectorSubcoreMesh(core_axis_name="c", subcore_axis_name="s"),
    scratch_types=[
        pltpu.VMEM((NCHUNK, K), jnp.int32),
        pltpu.VMEM((NBUF, K, D), jnp.float32),
    ] + [pltpu.SemaphoreType.DMA] * (2 * NBUF),
)(_body)


@jax.jit
def kernel(indices, embedding_table):
    idx = indices.reshape(NW, NCHUNK, K).astype(jnp.int32)
    out = _gather_call(idx, embedding_table)
    return out.reshape(16384, 26, D)


# speedup vs baseline: 5.7031x; 1.6872x over previous
import functools

import jax
import jax.numpy as jnp
from jax import lax
from jax.experimental import pallas as pl
from jax.experimental.pallas import tpu as pltpu
from jax.experimental.pallas import tpu_sc as plsc

NC = 2
NS = 16
NW = NC * NS

NSAMP = 16384
NCOL = 26
D = 128
S_PER_W = NSAMP // NW
SPC = 4
K = SPC * NCOL
NCHUNK = S_PER_W // SPC
NBUF = 4
NROUND = NCHUNK // NBUF


def _body(idx_hbm, table_hbm, out_hbm, idx_v, rows_v, *sems):
    sems_g, sems_w = sems[:NBUF], sems[NBUF:]
    wid = lax.axis_index("s") * NC + lax.axis_index("c")
    pltpu.sync_copy(idx_hbm.at[wid], idx_v)
    sample_base = wid * S_PER_W

    def start_gather(c, b):
        pltpu.async_copy(table_hbm.at[idx_v.at[c]], rows_v.at[b], sems_g[b])

    def wait_gather(b):
        pltpu.make_async_copy(
            table_hbm.at[idx_v.at[0]], rows_v.at[b], sems_g[b]).wait()

    def wait_write(b):
        pltpu.make_async_copy(
            rows_v.at[b].at[pl.ds(0, NCOL)], out_hbm.at[0], sems_w[b]).wait()

    def visit(g, b, do_wait_w, do_gather):
        bw = (b + 2) % NBUF
        if do_wait_w:
            for _ in range(SPC):
                wait_write(bw)
        if do_gather:
            start_gather(g + 2, bw)
        wait_gather(b)
        s0 = sample_base + g * SPC
        for i in range(SPC):
            pltpu.async_copy(rows_v.at[b].at[pl.ds(i * NCOL, NCOL)],
                             out_hbm.at[s0 + i], sems_w[b])

    start_gather(0, 0)
    start_gather(1, 1)

    visit(0, 0, False, True)
    visit(1, 1, False, True)
    visit(2, 2, True, True)
    visit(3, 3, True, True)

    def round_body(r, carry):
        g0 = r * NBUF
        for b in range(NBUF):
            visit(g0 + b, b, True, True)
        return carry

    lax.fori_loop(1, NROUND - 1, round_body, 0)

    g0 = (NROUND - 1) * NBUF
    visit(g0 + 0, 0, True, True)
    visit(g0 + 1, 1, True, True)
    visit(g0 + 2, 2, True, False)
    visit(g0 + 3, 3, True, False)

    for b in (2, 3):
        for _ in range(SPC):
            wait_write(b)


_gather_call = functools.partial(
    pl.kernel,
    out_type=jax.ShapeDtypeStruct((NSAMP, NCOL, D), jnp.float32),
    mesh=plsc.VectorSubcoreMesh(core_axis_name="c", subcore_axis_name="s"),
    scratch_types=[
        pltpu.VMEM((NCHUNK, K), jnp.int32),
        pltpu.VMEM((NBUF, K, D), jnp.float32),
    ] + [pltpu.SemaphoreType.DMA] * (2 * NBUF),
    compiler_params=pltpu.CompilerParams(use_tc_tiling_on_sc=True),
)(_body)


@jax.jit
def kernel(indices, embedding_table):
    idx = indices.reshape(NW, NCHUNK, K).astype(jnp.int32)
    return _gather_call(idx, embedding_table)


# speedup vs baseline: 11.7213x; 2.0552x over previous
import functools

import jax
import jax.numpy as jnp
from jax import lax
from jax.experimental import pallas as pl
from jax.experimental.pallas import tpu as pltpu
from jax.experimental.pallas import tpu_sc as plsc

NC = 2
NS = 16
NW = NC * NS

NSAMP = 16384
NCOL = 26
D = 128
S_PER_W = NSAMP // NW
GR = 128
KPC = S_PER_W // GR
NCHUNK = NCOL * KPC
NBUF = 4
NROUND = NCHUNK // NBUF


def _body(idx_hbm, table_hbm, out_hbm, idx_v, rows_v, *sems):
    sems_g, sems_w = sems[:NBUF], sems[NBUF:]
    wid = lax.axis_index("s") * NC + lax.axis_index("c")
    pltpu.sync_copy(idx_hbm.at[wid], idx_v)
    sample_base = wid * S_PER_W

    def start_gather(g, b):
        pltpu.async_copy(table_hbm.at[idx_v.at[g]], rows_v.at[b], sems_g[b])

    def wait_gather(b):
        pltpu.make_async_copy(
            table_hbm.at[idx_v.at[0]], rows_v.at[b], sems_g[b]).wait()

    def wait_write(b):
        pltpu.make_async_copy(
            rows_v.at[b], out_hbm.at[0].at[pl.ds(0, GR)], sems_w[b]).wait()

    def visit(g, b, do_wait_w, do_gather):
        bw = (b + 2) % NBUF
        if do_wait_w:
            wait_write(bw)
        if do_gather:
            start_gather(g + 2, bw)
        wait_gather(b)
        c = g // KPC
        j0 = sample_base + (g % KPC) * GR
        pltpu.async_copy(rows_v.at[b], out_hbm.at[c].at[pl.ds(j0, GR)],
                         sems_w[b])

    start_gather(0, 0)
    start_gather(1, 1)

    visit(0, 0, False, True)
    visit(1, 1, False, True)
    visit(2, 2, True, True)
    visit(3, 3, True, True)

    def round_body(r, carry):
        g0 = r * NBUF
        for b in range(NBUF):
            visit(g0 + b, b, True, True)
        return carry

    lax.fori_loop(1, NROUND - 1, round_body, 0)

    g0 = (NROUND - 1) * NBUF
    visit(g0 + 0, 0, True, True)
    visit(g0 + 1, 1, True, True)
    visit(g0 + 2, 2, True, False)
    visit(g0 + 3, 3, True, False)

    for b in (2, 3):
        wait_write(b)


_gather_call = functools.partial(
    pl.kernel,
    out_type=jax.ShapeDtypeStruct((NCOL, NSAMP, D), jnp.float32),
    mesh=plsc.VectorSubcoreMesh(core_axis_name="c", subcore_axis_name="s"),
    scratch_types=[
        pltpu.VMEM((NCHUNK, GR), jnp.int32),
        pltpu.VMEM((NBUF, GR, D), jnp.float32),
    ] + [pltpu.SemaphoreType.DMA] * (2 * NBUF),
    compiler_params=pltpu.CompilerParams(use_tc_tiling_on_sc=True),
)(_body)


@jax.jit
def kernel(indices, embedding_table):
    idx = indices.astype(jnp.int32)
    idx = idx.reshape(NW, S_PER_W, NCOL).transpose(0, 2, 1)
    idx = idx.reshape(NW, NCHUNK, GR)
    out = _gather_call(idx, embedding_table)
    return out.transpose(1, 0, 2)


# speedup vs baseline: 11.7314x; 1.0009x over previous
import functools

import jax
import jax.numpy as jnp
from jax import lax
from jax.experimental import pallas as pl
from jax.experimental.pallas import tpu as pltpu
from jax.experimental.pallas import tpu_sc as plsc

NC = 2
NS = 16
NW = NC * NS

NSAMP = 16384
NCOL = 26
D = 128
S_PER_W = NSAMP // NW
GR = 128
KPC = S_PER_W // GR
NCHUNK = NCOL * KPC
NBUF = 4
LA = 3
NROUND = NCHUNK // NBUF


def _body(idx_hbm, table_hbm, out_hbm, idx_v, rows_v, *sems):
    sems_g, sems_w = sems[:NBUF], sems[NBUF:]
    wid = lax.axis_index("s") * NC + lax.axis_index("c")
    pltpu.sync_copy(idx_hbm.at[wid], idx_v)
    sample_base = wid * S_PER_W

    def start_gather(g, b):
        pltpu.async_copy(table_hbm.at[idx_v.at[g]], rows_v.at[b], sems_g[b])

    def wait_gather(b):
        pltpu.make_async_copy(
            table_hbm.at[idx_v.at[0]], rows_v.at[b], sems_g[b]).wait()

    def wait_write(b):
        pltpu.make_async_copy(
            rows_v.at[b], out_hbm.at[0].at[pl.ds(0, GR)], sems_w[b]).wait()

    def visit(g, b, do_wait_w, do_gather):
        bw = (b + LA) % NBUF
        if do_wait_w:
            wait_write(bw)
        if do_gather:
            start_gather(g + LA, bw)
        wait_gather(b)
        c = g // KPC
        j0 = sample_base + (g % KPC) * GR
        pltpu.async_copy(rows_v.at[b], out_hbm.at[c].at[pl.ds(j0, GR)],
                         sems_w[b])

    for k in range(LA):
        start_gather(k, k)

    for b in range(NBUF):
        visit(b, b, b >= NBUF - LA, True)

    def round_body(r, carry):
        g0 = r * NBUF
        for b in range(NBUF):
            visit(g0 + b, b, True, True)
        return carry

    lax.fori_loop(1, NROUND - 1, round_body, 0)

    g0 = (NROUND - 1) * NBUF
    for b in range(NBUF):
        visit(g0 + b, b, True, g0 + b + LA < NCHUNK)

    for b in range(LA, NBUF):
        wait_write(b)


_gather_call = functools.partial(
    pl.kernel,
    out_type=jax.ShapeDtypeStruct((NCOL, NSAMP, D), jnp.float32),
    mesh=plsc.VectorSubcoreMesh(core_axis_name="c", subcore_axis_name="s"),
    scratch_types=[
        pltpu.VMEM((NCHUNK, GR), jnp.int32),
        pltpu.VMEM((NBUF, GR, D), jnp.float32),
    ] + [pltpu.SemaphoreType.DMA] * (2 * NBUF),
    compiler_params=pltpu.CompilerParams(use_tc_tiling_on_sc=True),
)(_body)


@jax.jit
def kernel(indices, embedding_table):
    idx = indices.astype(jnp.int32)
    idx = idx.reshape(NW, S_PER_W, NCOL).transpose(0, 2, 1)
    idx = idx.reshape(NW, NCHUNK, GR)
    out = _gather_call(idx, embedding_table)
    return out.transpose(1, 0, 2)


# speedup vs baseline: 11.9717x; 1.0205x over previous
import functools

import jax
import jax.numpy as jnp
from jax import lax
from jax.experimental import pallas as pl
from jax.experimental.pallas import tpu as pltpu
from jax.experimental.pallas import tpu_sc as plsc

NC = 2
NS = 16
NW = NC * NS

NSAMP = 16384
NCOL = 26
D = 128
S_PER_W = NSAMP // NW
GR = 128
KPC = S_PER_W // GR
NCHUNK = NCOL * KPC
NBUF = 4
LA = 2
NROUND = NCHUNK // NBUF


def _body(idx_hbm, table_hbm, out_hbm, idx_v, rows_v, *sems):
    sems_g, sems_w = sems[:NBUF], sems[NBUF:]
    wid = lax.axis_index("s") * NC + lax.axis_index("c")
    sample_base = wid * S_PER_W
    pltpu.sync_copy(idx_hbm.at[:, pl.ds(sample_base, S_PER_W)], idx_v)

    def idx_vec(g):
        return idx_v.at[g // KPC].at[pl.ds((g % KPC) * GR, GR)]

    def start_gather(g, b):
        pltpu.async_copy(table_hbm.at[idx_vec(g)], rows_v.at[b], sems_g[b])

    def wait_gather(b):
        pltpu.make_async_copy(
            table_hbm.at[idx_vec(0)], rows_v.at[b], sems_g[b]).wait()

    def wait_write(b):
        pltpu.make_async_copy(
            rows_v.at[b], out_hbm.at[0].at[pl.ds(0, GR)], sems_w[b]).wait()

    def visit(g, b, do_wait_w, do_gather):
        bw = (b + LA) % NBUF
        if do_wait_w:
            wait_write(bw)
        if do_gather:
            start_gather(g + LA, bw)
        wait_gather(b)
        c = g // KPC
        j0 = sample_base + (g % KPC) * GR
        pltpu.async_copy(rows_v.at[b], out_hbm.at[c].at[pl.ds(j0, GR)],
                         sems_w[b])

    for k in range(LA):
        start_gather(k, k)

    for b in range(NBUF):
        visit(b, b, b >= NBUF - LA, True)

    def round_body(r, carry):
        g0 = r * NBUF
        for b in range(NBUF):
            visit(g0 + b, b, True, True)
        return carry

    lax.fori_loop(1, NROUND - 1, round_body, 0)

    g0 = (NROUND - 1) * NBUF
    for b in range(NBUF):
        visit(g0 + b, b, True, g0 + b + LA < NCHUNK)

    for b in range(LA, NBUF):
        wait_write(b)


_gather_call = functools.partial(
    pl.kernel,
    out_type=jax.ShapeDtypeStruct((NCOL, NSAMP, D), jnp.float32),
    mesh=plsc.VectorSubcoreMesh(core_axis_name="c", subcore_axis_name="s"),
    scratch_types=[
        pltpu.VMEM((NCOL, S_PER_W), jnp.int32),
        pltpu.VMEM((NBUF, GR, D), jnp.float32),
    ] + [pltpu.SemaphoreType.DMA] * (2 * NBUF),
    compiler_params=pltpu.CompilerParams(use_tc_tiling_on_sc=True),
)(_body)


@jax.jit
def kernel(indices, embedding_table):
    idx = indices.astype(jnp.int32).T
    out = _gather_call(idx, embedding_table)
    return out.transpose(1, 0, 2)
